# baseline (speedup 1.0000x reference)
import jax
import jax.numpy as jnp
from jax import lax
from jax.experimental import pallas as pl
from jax.experimental.pallas import tpu as pltpu
from jax.experimental.pallas import tpu_sc as plsc

NUM_OPS = 1000000
EMBED_D = 32
N = 16384

_NC = 2
_NS = 16
_NW = _NC * _NS
_BPW = N // _NW
_NQ = (NUM_OPS + 127) // 128
_QPW = (_NQ + _NW - 1) // _NW
_QBLK = 4
_RING = 4
_NSUP = 16
_NBLK = _NSUP * _RING
_MAXF = _NQ - _QBLK


def _stream_body(idx_hbm, table3_hbm, scr_hbm, idx_v, i_own, p_own, blk0, blk1,
                 blk2, blk3, row_v, sem_b0, sem_b1, sem_b2, sem_b3, sem_r):
  wid = lax.axis_index("s") * _NC + lax.axis_index("c")
  qlo = wid * _QPW
  qhi = jnp.minimum(qlo + _QPW, _NQ)
  pltpu.sync_copy(idx_hbm, idx_v)

  lanes = lax.iota(jnp.int32, 16)
  trv = lax.shift_right_logical(lanes, 3)
  sv = lax.bitwise_and(lanes, 7)

  def scan_body(c, cnt):
    vec = idx_v[pl.ds(c * 16, 16)]
    qv = lax.shift_right_logical(vec, 7)
    m = jnp.logical_and(qv >= qlo, qv < qhi)
    mi = m.astype(jnp.int32)
    incl = plsc.cumsum(mi)
    offv = (incl - mi) + cnt
    plsc.store_scatter(i_own, [offv], vec, mask=m)
    plsc.store_scatter(p_own, [offv], c * 16 + lanes, mask=m)
    return cnt + incl[15]

  cnt = lax.fori_loop(0, N // 16, scan_body, 0)
  nch = lax.shift_right_logical(cnt + 15, 4)

  sems = [sem_b0, sem_b1, sem_b2, sem_b3]
  blks = [blk0, blk1, blk2, blk3]

  def fetch(b, slot):
    bs = jnp.minimum(qlo + b * _QBLK, _MAXF)
    for tr in range(4):
      pltpu.async_copy(
          table3_hbm.at[tr, :, pl.ds(bs * 128, _QBLK * 128)],
          blks[slot].at[tr],
          sems[slot],
      )

  for b in range(_RING - 1):
    fetch(b, b)

  def super_body(sup, hc):
    for rb in range(_RING):
      b = sup * _RING + rb
      bs_lo = qlo + b * _QBLK
      bs_hi = jnp.minimum(bs_lo + _QBLK, qhi)
      bsf = jnp.minimum(bs_lo, _MAXF)
      for tr in range(4):
        pltpu.make_async_copy(
            table3_hbm.at[tr, :, pl.ds(0, _QBLK * 128)],
            blks[rb].at[tr],
            sems[rb],
        ).wait()

      @pl.when(b + _RING - 1 < _NBLK)
      def _(b=b, rb=rb):
        fetch(b + _RING - 1, (rb + _RING - 1) % _RING)

      def chunk_body(c, hc, bs_lo=bs_lo, bs_hi=bs_hi, bsf=bsf, rb=rb):
        iv = i_own[pl.ds(c * 16, 16)]
        pv = p_own[pl.ds(c * 16, 16)]
        qv = lax.shift_right_logical(iv, 7)
        m = jnp.logical_and(
            jnp.logical_and(qv >= bs_lo, qv < bs_hi), (c * 16 + lanes) < cnt
        )
        n = plsc.all_reduce_population_count(m)[0]

        def hbody(h, st, iv=iv, pv=pv, bsf=bsf, rb=rb):
          hc2, m2 = st
          lanev = plsc.all_reduce_ffs(m2)
          hitl = lanes == lanev
          i_s = jnp.sum(jnp.where(hitl, iv, 0))
          p_s = jnp.sum(jnp.where(hitl, pv, 0))
          q_s = lax.shift_right_logical(i_s, 7)
          r_s = lax.bitwise_and(i_s, 127)
          lv_loc = jnp.broadcast_to((q_s - bsf) * 128 + r_s, (16,))
          v0 = plsc.load_gather(blks[rb], [trv, sv, lv_loc])
          v1 = plsc.load_gather(blks[rb], [trv + 2, sv, lv_loc])
          slot = lax.bitwise_and(hc2, 15)

          @pl.when(jnp.logical_and(slot == 0, hc2 > 0))
          def _():
            pltpu.make_async_copy(
                scr_hbm.at[pl.ds(0, 512)], row_v, sem_r
            ).wait()

          row_v[pl.ds(slot * 32, 16)] = v0
          row_v[pl.ds(slot * 32 + 16, 16)] = v1
          pltpu.async_copy(
              row_v.at[pl.ds(slot * 32, 32)],
              scr_hbm.at[pl.ds(p_s * 32, 32)],
              sem_r,
          )
          return hc2 + 1, jnp.logical_and(m2, lanes != lanev)

        hc, _ = lax.fori_loop(0, n, hbody, (hc, m))
        return hc

      hc = lax.fori_loop(0, nch, chunk_body, hc)
    return hc

  hc = lax.fori_loop(0, _NSUP, super_body, 0)

  def drain_body(d, carry):
    pltpu.make_async_copy(
        scr_hbm.at[pl.ds(0, 32)], row_v.at[pl.ds(0, 32)], sem_r
    ).wait()
    return carry

  lax.fori_loop(0, lax.bitwise_and(hc, 15), drain_body, 0)


def _transpose_body(scr_hbm, out3_hbm, buf_v, big, sem):
  wid = lax.axis_index("s") * _NC + lax.axis_index("c")
  base = wid * _BPW
  pltpu.sync_copy(scr_hbm.at[pl.ds(base * 32, _BPW * 32)], buf_v)

  lanes = lax.iota(jnp.int32, 16)
  trv = lax.shift_right_logical(lanes, 3)
  sv = lax.bitwise_and(lanes, 7)

  def body(j, carry):
    src = j * 32 + lanes
    v0 = plsc.load_gather(buf_v, [src])
    v1 = plsc.load_gather(buf_v, [src + 16])
    tcv = jnp.broadcast_to(lax.shift_right_logical(j, 7), (16,))
    lv = jnp.broadcast_to(lax.bitwise_and(j, 127), (16,))
    plsc.store_scatter(big, [trv, tcv, sv, lv], v0)
    plsc.store_scatter(big, [trv + 2, tcv, sv, lv], v1)
    return carry

  lax.fori_loop(0, _BPW, body, 0)
  for tr in range(4):
    for tc in range(4):
      pltpu.sync_copy(
          big.at[tr, tc], out3_hbm.at[tr, :, pl.ds(base + tc * 128, 128)]
      )


_mesh = plsc.VectorSubcoreMesh(core_axis_name="c", subcore_axis_name="s")

_params = pltpu.CompilerParams(
    disable_bounds_checks=True, needs_layout_passes=False
)

_stream = pl.kernel(
    _stream_body,
    out_type=jax.ShapeDtypeStruct((N * EMBED_D,), jnp.float32),
    mesh=_mesh,
    scratch_types=[
        pltpu.VMEM((N,), jnp.int32),
        pltpu.VMEM((N,), jnp.int32),
        pltpu.VMEM((N,), jnp.int32),
        pltpu.VMEM((4, 8, _QBLK * 128), jnp.float32),
        pltpu.VMEM((4, 8, _QBLK * 128), jnp.float32),
        pltpu.VMEM((4, 8, _QBLK * 128), jnp.float32),
        pltpu.VMEM((4, 8, _QBLK * 128), jnp.float32),
        pltpu.VMEM((512,), jnp.float32),
        pltpu.SemaphoreType.DMA,
        pltpu.SemaphoreType.DMA,
        pltpu.SemaphoreType.DMA,
        pltpu.SemaphoreType.DMA,
        pltpu.SemaphoreType.DMA,
    ],
    compiler_params=_params,
)

_transpose = pl.kernel(
    _transpose_body,
    out_type=jax.ShapeDtypeStruct((4, 8, N), jnp.float32),
    mesh=_mesh,
    scratch_types=[
        pltpu.VMEM((_BPW * 32,), jnp.float32),
        pltpu.VMEM((4, 4, 8, 128), jnp.float32),
        pltpu.SemaphoreType.DMA,
    ],
    compiler_params=_params,
)


@jax.jit
def kernel(indices, table):
  table3 = table.T.reshape(4, 8, NUM_OPS)
  scr = _stream(indices, table3)
  out3 = _transpose(scr)
  return out3.reshape(EMBED_D, N).T


# speedup vs baseline: 1.0471x; 1.0471x over previous
import jax
import jax.numpy as jnp
from jax import lax
from jax.experimental import pallas as pl
from jax.experimental.pallas import tpu as pltpu
from jax.experimental.pallas import tpu_sc as plsc

NUM_OPS = 1000000
EMBED_D = 32
N = 16384

_NC = 2
_NS = 16
_NW = _NC * _NS
_BPW = N // _NW
_NQ = (NUM_OPS + 127) // 128
_QPW = (_NQ + _NW - 1) // _NW
_QBLK = 8
_NBLK = (_QPW + _QBLK - 1) // _QBLK
_MAXF = _NQ - _QBLK


def _stream_body(idx_hbm, table3_hbm, scr_hbm, idx_v, i_own, p_own, blk0, blk1,
                 row_v, sem_i, sem_b0, sem_b1, sem_r):
  wid = lax.axis_index("s") * _NC + lax.axis_index("c")
  qlo = wid * _QPW
  qhi = jnp.minimum(qlo + _QPW, _NQ)

  sems = [sem_b0, sem_b1]
  blks = [blk0, blk1]

  def fetch(b):
    bs = jnp.minimum(qlo + b * _QBLK, _MAXF)
    for tr in range(4):
      pltpu.async_copy(
          table3_hbm.at[tr, :, pl.ds(bs * 128, _QBLK * 128)],
          blks[b % 2].at[tr],
          sems[b % 2],
      )

  fetch(0)
  fetch(1)
  pltpu.sync_copy(idx_hbm, idx_v)

  lanes = lax.iota(jnp.int32, 16)
  trv = lax.shift_right_logical(lanes, 3)
  sv = lax.bitwise_and(lanes, 7)

  def scan_body(c, cnt):
    vec = idx_v[pl.ds(c * 16, 16)]
    qv = lax.shift_right_logical(vec, 7)
    m = jnp.logical_and(qv >= qlo, qv < qhi)
    n = plsc.all_reduce_population_count(m)[0]

    @pl.when(n > 0)
    def _():
      mi = m.astype(jnp.int32)
      incl = plsc.cumsum(mi)
      offv = (incl - mi) + cnt
      plsc.store_scatter(i_own, [offv], vec, mask=m)
      plsc.store_scatter(p_own, [offv], c * 16 + lanes, mask=m)

    return cnt + n

  cnt = lax.fori_loop(0, N // 16, scan_body, 0)
  nch = lax.shift_right_logical(cnt + 15, 4)

  hc = 0

  for b in range(_NBLK):
    bs_lo = qlo + b * _QBLK
    bs_hi = jnp.minimum(bs_lo + _QBLK, qhi)
    bsf = jnp.minimum(bs_lo, _MAXF)
    for tr in range(4):
      pltpu.make_async_copy(
          table3_hbm.at[tr, :, pl.ds(0, _QBLK * 128)],
          blks[b % 2].at[tr],
          sems[b % 2],
      ).wait()

    def chunk_body(c, hc, bs_lo=bs_lo, bs_hi=bs_hi, bsf=bsf, b=b):
      iv = i_own[pl.ds(c * 16, 16)]
      qv = lax.shift_right_logical(iv, 7)
      m = jnp.logical_and(
          jnp.logical_and(qv >= bs_lo, qv < bs_hi), (c * 16 + lanes) < cnt
      )
      n = plsc.all_reduce_population_count(m)[0]

      def hbody(h, st, iv=iv, c=c, bsf=bsf, b=b):
        hc2, m2 = st
        pv = p_own[pl.ds(c * 16, 16)]
        lanev = plsc.all_reduce_ffs(m2)
        hitl = lanes == lanev
        i_s = jnp.sum(jnp.where(hitl, iv, 0))
        p_s = jnp.sum(jnp.where(hitl, pv, 0))
        q_s = lax.shift_right_logical(i_s, 7)
        r_s = lax.bitwise_and(i_s, 127)
        lv_loc = jnp.broadcast_to((q_s - bsf) * 128 + r_s, (16,))
        v0 = plsc.load_gather(blks[b % 2], [trv, sv, lv_loc])
        v1 = plsc.load_gather(blks[b % 2], [trv + 2, sv, lv_loc])
        slot = lax.bitwise_and(hc2, 15)

        @pl.when(jnp.logical_and(slot == 0, hc2 > 0))
        def _():
          pltpu.make_async_copy(scr_hbm.at[pl.ds(0, 512)], row_v, sem_r).wait()

        row_v[pl.ds(slot * 32, 16)] = v0
        row_v[pl.ds(slot * 32 + 16, 16)] = v1
        pltpu.async_copy(
            row_v.at[pl.ds(slot * 32, 32)],
            scr_hbm.at[pl.ds(p_s * 32, 32)],
            sem_r,
        )
        return hc2 + 1, jnp.logical_and(m2, lanes != lanev)

      hc, _ = lax.fori_loop(0, n, hbody, (hc, m))
      return hc

    hc = lax.fori_loop(0, nch, chunk_body, hc)
    if b + 2 < _NBLK:
      fetch(b + 2)

  def drain_body(d, carry):
    pltpu.make_async_copy(
        scr_hbm.at[pl.ds(0, 32)], row_v.at[pl.ds(0, 32)], sem_r
    ).wait()
    return carry

  lax.fori_loop(0, lax.bitwise_and(hc, 15), drain_body, 0)


def _transpose_body(scr_hbm, out3_hbm, buf_v, big, sem):
  wid = lax.axis_index("s") * _NC + lax.axis_index("c")
  base = wid * _BPW
  pltpu.sync_copy(scr_hbm.at[pl.ds(base * 32, _BPW * 32)], buf_v)

  lanes = lax.iota(jnp.int32, 16)
  trv = lax.shift_right_logical(lanes, 3)
  sv = lax.bitwise_and(lanes, 7)

  def body(g, carry):
    j = g * 2
    tcv = jnp.broadcast_to(lax.shift_right_logical(j, 7), (16,))
    lv = jnp.broadcast_to(lax.bitwise_and(j, 127), (16,))
    src = j * 32 + lanes
    for d in range(2):
      v0 = plsc.load_gather(buf_v, [src + d * 32])
      v1 = plsc.load_gather(buf_v, [src + d * 32 + 16])
      plsc.store_scatter(big, [trv, tcv, sv, lv + d], v0)
      plsc.store_scatter(big, [trv + 2, tcv, sv, lv + d], v1)
    return carry

  lax.fori_loop(0, _BPW // 2, body, 0)
  for tr in range(4):
    for tc in range(4):
      pltpu.sync_copy(
          big.at[tr, tc], out3_hbm.at[tr, :, pl.ds(base + tc * 128, 128)]
      )


_mesh = plsc.VectorSubcoreMesh(core_axis_name="c", subcore_axis_name="s")

_params = pltpu.CompilerParams(
    disable_bounds_checks=True, needs_layout_passes=False
)

_stream = pl.kernel(
    _stream_body,
    out_type=jax.ShapeDtypeStruct((N * EMBED_D,), jnp.float32),
    mesh=_mesh,
    scratch_types=[
        pltpu.VMEM((N,), jnp.int32),
        pltpu.VMEM((N,), jnp.int32),
        pltpu.VMEM((N,), jnp.int32),
        pltpu.VMEM((4, 8, _QBLK * 128), jnp.float32),
        pltpu.VMEM((4, 8, _QBLK * 128), jnp.float32),
        pltpu.VMEM((512,), jnp.float32),
        pltpu.SemaphoreType.DMA,
        pltpu.SemaphoreType.DMA,
        pltpu.SemaphoreType.DMA,
        pltpu.SemaphoreType.DMA,
    ],
    compiler_params=_params,
)

_transpose = pl.kernel(
    _transpose_body,
    out_type=jax.ShapeDtypeStruct((4, 8, N), jnp.float32),
    mesh=_mesh,
    scratch_types=[
        pltpu.VMEM((_BPW * 32,), jnp.float32),
        pltpu.VMEM((4, 4, 8, 128), jnp.float32),
        pltpu.SemaphoreType.DMA,
    ],
    compiler_params=_params,
)


@jax.jit
def kernel(indices, table):
  table3 = table.T.reshape(4, 8, NUM_OPS)
  scr = _stream(indices, table3)
  out3 = _transpose(scr)
  return out3.reshape(EMBED_D, N).T


# speedup vs baseline: 1.1442x; 1.0927x over previous
import jax
import jax.numpy as jnp
from jax import lax
from jax.experimental import pallas as pl
from jax.experimental.pallas import tpu as pltpu
from jax.experimental.pallas import tpu_sc as plsc

NUM_OPS = 1000000
EMBED_D = 32
N = 16384

_NC = 2
_NS = 16
_NW = _NC * _NS
_BPW = N // _NW
_NQ = (NUM_OPS + 127) // 128
_QPW = (_NQ + _NW - 1) // _NW
_QBLK = 8
_NBLK = (_QPW + _QBLK - 1) // _QBLK
_MAXF = _NQ - _QBLK


def _stream_body(idx_hbm, table3_hbm, scr_hbm, idx_v, i_own, p_own, blk0, blk1,
                 row_v, sem_i, sem_b0, sem_b1, sem_r):
  wid = lax.axis_index("s") * _NC + lax.axis_index("c")
  qlo = wid * _QPW
  qhi = jnp.minimum(qlo + _QPW, _NQ)

  sems = [sem_b0, sem_b1]
  blks = [blk0, blk1]

  def fetch(b):
    bs = jnp.minimum(qlo + b * _QBLK, _MAXF)
    for tr in range(4):
      pltpu.async_copy(
          table3_hbm.at[tr, :, pl.ds(bs * 128, _QBLK * 128)],
          blks[b % 2].at[tr],
          sems[b % 2],
      )

  fetch(0)
  fetch(1)
  pltpu.sync_copy(idx_hbm, idx_v)

  lanes = lax.iota(jnp.int32, 16)
  trv = lax.shift_right_logical(lanes, 3)
  sv = lax.bitwise_and(lanes, 7)

  def scan_body(c, cnt):
    vec = idx_v[pl.ds(c * 16, 16)]
    qv = lax.shift_right_logical(vec, 7)
    m = jnp.logical_and(qv >= qlo, qv < qhi)
    mi = m.astype(jnp.int32)
    incl = plsc.cumsum(mi)
    offv = (incl - mi) + cnt
    plsc.store_scatter(i_own, [offv], vec, mask=m)
    plsc.store_scatter(p_own, [offv], c * 16 + lanes, mask=m)
    return cnt + incl[15]

  cnt = lax.fori_loop(0, N // 16, scan_body, 0)
  nch = lax.shift_right_logical(cnt + 15, 4)

  hc = 0

  for b in range(_NBLK):
    bs_lo = qlo + b * _QBLK
    bs_hi = jnp.minimum(bs_lo + _QBLK, qhi)
    bsf = jnp.minimum(bs_lo, _MAXF)
    for tr in range(4):
      pltpu.make_async_copy(
          table3_hbm.at[tr, :, pl.ds(0, _QBLK * 128)],
          blks[b % 2].at[tr],
          sems[b % 2],
      ).wait()

    def chunk_body(c, hc, bs_lo=bs_lo, bs_hi=bs_hi, bsf=bsf, b=b):
      iv = i_own[pl.ds(c * 16, 16)]
      qv = lax.shift_right_logical(iv, 7)
      m = jnp.logical_and(
          jnp.logical_and(qv >= bs_lo, qv < bs_hi), (c * 16 + lanes) < cnt
      )
      n = plsc.all_reduce_population_count(m)[0]

      def hbody(h, st, iv=iv, c=c, bsf=bsf, b=b):
        hc2, m2 = st
        pv = p_own[pl.ds(c * 16, 16)]
        lanev = plsc.all_reduce_ffs(m2)
        hitl = lanes == lanev
        i_s = jnp.sum(jnp.where(hitl, iv, 0))
        p_s = jnp.sum(jnp.where(hitl, pv, 0))
        q_s = lax.shift_right_logical(i_s, 7)
        r_s = lax.bitwise_and(i_s, 127)
        lv_loc = jnp.broadcast_to((q_s - bsf) * 128 + r_s, (16,))
        v0 = plsc.load_gather(blks[b % 2], [trv, sv, lv_loc])
        v1 = plsc.load_gather(blks[b % 2], [trv + 2, sv, lv_loc])
        slot = lax.bitwise_and(hc2, 15)

        @pl.when(jnp.logical_and(slot == 0, hc2 > 0))
        def _():
          pltpu.make_async_copy(scr_hbm.at[pl.ds(0, 512)], row_v, sem_r).wait()

        row_v[pl.ds(slot * 32, 16)] = v0
        row_v[pl.ds(slot * 32 + 16, 16)] = v1
        pltpu.async_copy(
            row_v.at[pl.ds(slot * 32, 32)],
            scr_hbm.at[pl.ds(p_s * 32, 32)],
            sem_r,
        )
        return hc2 + 1, jnp.logical_and(m2, lanes != lanev)

      hc, _ = lax.fori_loop(0, n, hbody, (hc, m))
      return hc

    hc = lax.fori_loop(0, nch, chunk_body, hc)
    if b + 2 < _NBLK:
      fetch(b + 2)

  def drain_body(d, carry):
    pltpu.make_async_copy(
        scr_hbm.at[pl.ds(0, 32)], row_v.at[pl.ds(0, 32)], sem_r
    ).wait()
    return carry

  lax.fori_loop(0, lax.bitwise_and(hc, 15), drain_body, 0)


def _transpose_body(scr_hbm, out3_hbm, buf_v, big, sem):
  wid = lax.axis_index("s") * _NC + lax.axis_index("c")
  base = wid * _BPW
  pltpu.sync_copy(scr_hbm.at[pl.ds(base * 32, _BPW * 32)], buf_v)

  lanes = lax.iota(jnp.int32, 16)
  trv = lax.shift_right_logical(lanes, 3)
  sv = lax.bitwise_and(lanes, 7)

  def body(g, carry):
    j = g * 2
    tcv = jnp.broadcast_to(lax.shift_right_logical(j, 7), (16,))
    lv = jnp.broadcast_to(lax.bitwise_and(j, 127), (16,))
    src = j * 32 + lanes
    for d in range(2):
      v0 = plsc.load_gather(buf_v, [src + d * 32])
      v1 = plsc.load_gather(buf_v, [src + d * 32 + 16])
      plsc.store_scatter(big, [trv, tcv, sv, lv + d], v0)
      plsc.store_scatter(big, [trv + 2, tcv, sv, lv + d], v1)
    return carry

  lax.fori_loop(0, _BPW // 2, body, 0)
  for tr in range(4):
    for tc in range(4):
      pltpu.sync_copy(
          big.at[tr, tc], out3_hbm.at[tr, :, pl.ds(base + tc * 128, 128)]
      )


_mesh = plsc.VectorSubcoreMesh(core_axis_name="c", subcore_axis_name="s")

_params = pltpu.CompilerParams(
    disable_bounds_checks=True, needs_layout_passes=False
)

_stream = pl.kernel(
    _stream_body,
    out_type=jax.ShapeDtypeStruct((N * EMBED_D,), jnp.float32),
    mesh=_mesh,
    scratch_types=[
        pltpu.VMEM((N,), jnp.int32),
        pltpu.VMEM((N,), jnp.int32),
        pltpu.VMEM((N,), jnp.int32),
        pltpu.VMEM((4, 8, _QBLK * 128), jnp.float32),
        pltpu.VMEM((4, 8, _QBLK * 128), jnp.float32),
        pltpu.VMEM((512,), jnp.float32),
        pltpu.SemaphoreType.DMA,
        pltpu.SemaphoreType.DMA,
        pltpu.SemaphoreType.DMA,
        pltpu.SemaphoreType.DMA,
    ],
    compiler_params=_params,
)

_transpose = pl.kernel(
    _transpose_body,
    out_type=jax.ShapeDtypeStruct((4, 8, N), jnp.float32),
    mesh=_mesh,
    scratch_types=[
        pltpu.VMEM((_BPW * 32,), jnp.float32),
        pltpu.VMEM((4, 4, 8, 128), jnp.float32),
        pltpu.SemaphoreType.DMA,
    ],
    compiler_params=_params,
)


@jax.jit
def kernel(indices, table):
  table3 = table.T.reshape(4, 8, NUM_OPS)
  scr = _stream(indices, table3)
  out3 = _transpose(scr)
  return out3.reshape(EMBED_D, N).T
